# baseline (speedup 1.0000x reference)
import functools

import jax
import jax.numpy as jnp
from jax import lax
from jax.experimental import pallas as pl
from jax.experimental.pallas import tpu as pltpu
from jax.experimental.pallas import tpu_sc as plsc

N = 8192
D = 768
E = 64
F = 1024
RB = 512
NRB = N // RB
TM = 64
NPAD = 12288
NT = NPAD // TM

SC_CORES = 2
SC_SUBCORES = 16
NW = SC_CORES * SC_SUBCORES
TPW = N // NW
CB = 64
NCH = TPW // CB


def _router_body(x_ref, ws_ref, bs_ref, routes_ref, ranks_ref, pmax_ref,
                 aux_ref, starts_ref, te_ref, ut_ref,
                 tri_ref, counts_ref, colsum_ref):
    i = pl.program_id(0)

    @pl.when(i == 0)
    def _init():
        counts_ref[...] = jnp.zeros_like(counts_ref)
        colsum_ref[...] = jnp.zeros_like(colsum_ref)
        ri = lax.broadcasted_iota(jnp.int32, (RB, RB), 0)
        rj = lax.broadcasted_iota(jnp.int32, (RB, RB), 1)
        tri_ref[...] = (rj <= ri).astype(jnp.bfloat16)

    x = x_ref[...]
    logits = jnp.dot(x, ws_ref[...], preferred_element_type=jnp.float32)
    logits = logits + bs_ref[...]

    m = jnp.max(logits, axis=1, keepdims=True)
    p = jnp.exp(logits - m)
    s = jnp.sum(p, axis=1, keepdims=True)
    probs = p / s
    pmax = 1.0 / s[:, 0]

    iota_e = lax.broadcasted_iota(jnp.int32, (RB, E), 1)
    is_max = logits == m
    routes = jnp.min(jnp.where(is_max, iota_e, E), axis=1)

    onehot = (iota_e == routes[:, None]).astype(jnp.float32)

    csum = jnp.dot(tri_ref[...], onehot.astype(jnp.bfloat16),
                   preferred_element_type=jnp.float32)
    rank_in_blk = jnp.sum(csum * onehot, axis=1) - 1.0
    running = counts_ref[...]
    rank = rank_in_blk + jnp.sum(onehot * running, axis=1)

    cnt_blk = jnp.sum(onehot, axis=0, keepdims=True)
    counts_ref[...] = running + cnt_blk
    colsum_ref[...] = colsum_ref[...] + jnp.sum(probs, axis=0, keepdims=True)

    routes_ref[...] = routes.reshape(1, 1, RB)
    ranks_ref[...] = rank.astype(jnp.int32).reshape(1, 1, RB)
    pmax_ref[...] = pmax.reshape(1, 1, RB)

    @pl.when(i == NRB - 1)
    def _fin():
        counts = counts_ref[...]
        aux_ref[...] = ((E / N) * jnp.sum(counts * colsum_ref[...])
                        ).reshape(1, 1)
        padded = jnp.ceil(counts / TM) * TM
        ei = lax.broadcasted_iota(jnp.int32, (E, E), 0)
        ej = lax.broadcasted_iota(jnp.int32, (E, E), 1)
        triu = (ei <= ej).astype(jnp.bfloat16)
        ends = jnp.dot(padded.astype(jnp.bfloat16), triu,
                       preferred_element_type=jnp.float32)
        starts_ref[...] = (ends - padded).astype(jnp.int32)
        ends_last = jnp.max(ends, axis=1, keepdims=True)
        used = ends_last / TM
        ut_ref[...] = used.astype(jnp.int32)
        tgrid = lax.broadcasted_iota(jnp.int32, (NT, E), 0).astype(
            jnp.float32) * TM
        endsg = jnp.broadcast_to(ends, (NT, E))
        te_raw = jnp.sum((endsg <= tgrid).astype(jnp.float32), axis=1)
        te_last = jnp.minimum(
            jnp.sum((ends <= ends_last - TM).astype(jnp.float32)), E - 1.0)
        ti = lax.broadcasted_iota(jnp.int32, (NT,), 0).astype(jnp.float32)
        te = jnp.where(ti < used[0, 0], jnp.minimum(te_raw, E - 1.0), te_last)
        te_ref[...] = te.astype(jnp.int32).reshape(1, NT)


def _router(xf, Ws, bs):
    return pl.pallas_call(
        _router_body,
        grid=(NRB,),
        in_specs=[
            pl.BlockSpec((RB, D), lambda i: (i, 0)),
            pl.BlockSpec((D, E), lambda i: (0, 0)),
            pl.BlockSpec((1, E), lambda i: (0, 0)),
        ],
        out_specs=[
            pl.BlockSpec((1, 1, RB), lambda i: (i, 0, 0)),
            pl.BlockSpec((1, 1, RB), lambda i: (i, 0, 0)),
            pl.BlockSpec((1, 1, RB), lambda i: (i, 0, 0)),
            pl.BlockSpec((1, 1), lambda i: (0, 0)),
            pl.BlockSpec((1, E), lambda i: (0, 0)),
            pl.BlockSpec((1, NT), lambda i: (0, 0)),
            pl.BlockSpec((1, 1), lambda i: (0, 0)),
        ],
        out_shape=[
            jax.ShapeDtypeStruct((NRB, 1, RB), jnp.int32),
            jax.ShapeDtypeStruct((NRB, 1, RB), jnp.int32),
            jax.ShapeDtypeStruct((NRB, 1, RB), jnp.float32),
            jax.ShapeDtypeStruct((1, 1), jnp.float32),
            jax.ShapeDtypeStruct((1, E), jnp.int32),
            jax.ShapeDtypeStruct((1, NT), jnp.int32),
            jax.ShapeDtypeStruct((1, 1), jnp.int32),
        ],
        scratch_shapes=[
            pltpu.VMEM((RB, RB), jnp.bfloat16),
            pltpu.VMEM((1, E), jnp.float32),
            pltpu.VMEM((1, E), jnp.float32),
        ],
    )(xf, Ws, bs.reshape(1, E))


def _sc_mesh():
    return plsc.VectorSubcoreMesh(
        core_axis_name="c", subcore_axis_name="s",
        num_cores=SC_CORES, num_subcores=SC_SUBCORES)


_SC_PARAMS = pltpu.CompilerParams(needs_layout_passes=False)


def _sc_scatter_body(xf, routes, ranks, starts, xs, dest,
                     rows0, rows1, rr0, rr1, idx_v, starts_v,
                     sr0, sr1, sq0, sq1, sk0, sk1, sem):
    wid = lax.axis_index("s") * SC_CORES + lax.axis_index("c")
    wbase = wid * TPW
    rows = (rows0, rows1)
    rr = (rr0, rr1)
    srow = (sr0, sr1)
    srt = (sq0, sq1)
    srk = (sk0, sk1)
    pltpu.sync_copy(starts, starts_v)

    def start_in(c):
        b = c % 2
        base = wbase + c * CB
        return (
            pltpu.async_copy(xf.at[pl.ds(base, CB)], rows[b], srow[b]),
            pltpu.async_copy(routes.at[pl.ds(base, CB)], rr[b].at[0], srt[b]),
            pltpu.async_copy(ranks.at[pl.ds(base, CB)], rr[b].at[1], srk[b]),
        )

    h_in = start_in(0)
    for c in range(NCH):
        b = c % 2
        for h in h_in:
            h.wait()
        if c + 1 < NCH:
            h_in = start_in(c + 1)
        for k in range(CB // 16):
            r = rr[b][0, pl.ds(k * 16, 16)]
            s_r = plsc.load_gather(starts_v, [r])
            idx_v[c, pl.ds(k * 16, 16)] = s_r + rr[b][1, pl.ds(k * 16, 16)]
        pltpu.async_copy(rows[b], xs.at[idx_v.at[c]], sem).wait()
        pltpu.sync_copy(idx_v.at[c], dest.at[pl.ds(wbase + c * CB, CB)])


def _sc_scatter(xf, routes, ranks, starts):
    return pl.kernel(
        _sc_scatter_body,
        out_type=[
            jax.ShapeDtypeStruct((NPAD, D), jnp.float32),
            jax.ShapeDtypeStruct((N,), jnp.int32),
        ],
        mesh=_sc_mesh(),
        compiler_params=_SC_PARAMS,
        scratch_types=[
            pltpu.VMEM((CB, D), jnp.float32),
            pltpu.VMEM((CB, D), jnp.float32),
            pltpu.VMEM((2, CB), jnp.int32),
            pltpu.VMEM((2, CB), jnp.int32),
            pltpu.VMEM((NCH, CB), jnp.int32),
            pltpu.VMEM((E,), jnp.int32),
            pltpu.SemaphoreType.DMA,
            pltpu.SemaphoreType.DMA,
            pltpu.SemaphoreType.DMA,
            pltpu.SemaphoreType.DMA,
            pltpu.SemaphoreType.DMA,
            pltpu.SemaphoreType.DMA,
            pltpu.SemaphoreType.DMA,
        ],
    )(xf, routes, ranks, starts)


def _sc_gather_body(outp, dest, pmax, final,
                    rows_v, idx0, idx1, pm0, pm1,
                    si0, si1, sp0, sp1, sem):
    wid = lax.axis_index("s") * SC_CORES + lax.axis_index("c")
    wbase = wid * TPW
    idxb = (idx0, idx1)
    pmb = (pm0, pm1)
    sidx = (si0, si1)
    spm = (sp0, sp1)

    def start_idx(c):
        b = c % 2
        base = wbase + c * CB
        return (
            pltpu.async_copy(dest.at[pl.ds(base, CB)], idxb[b], sidx[b]),
            pltpu.async_copy(pmax.at[pl.ds(base, CB)], pmb[b], spm[b]),
        )

    h_in = start_idx(0)
    for c in range(NCH):
        b = c % 2
        for h in h_in:
            h.wait()
        if c + 1 < NCH:
            h_in = start_idx(c + 1)
        pltpu.async_copy(outp.at[idxb[b]], rows_v, sem).wait()

        def _scale(k, carry):
            pv = plsc.load_gather(pmb[b], [jnp.zeros((16,), jnp.int32) + k])
            for j in range(D // 16):
                rows_v[k, pl.ds(j * 16, 16)] = rows_v[k, pl.ds(j * 16, 16)] * pv
            return carry

        lax.fori_loop(0, CB, _scale, 0)
        pltpu.sync_copy(rows_v, final.at[pl.ds(wbase + c * CB, CB)])


def _sc_gather(outp, dest, pmax):
    return pl.kernel(
        _sc_gather_body,
        out_type=jax.ShapeDtypeStruct((N, D), jnp.float32),
        mesh=_sc_mesh(),
        compiler_params=_SC_PARAMS,
        scratch_types=[
            pltpu.VMEM((CB, D), jnp.float32),
            pltpu.VMEM((CB,), jnp.int32),
            pltpu.VMEM((CB,), jnp.int32),
            pltpu.VMEM((CB,), jnp.float32),
            pltpu.VMEM((CB,), jnp.float32),
            pltpu.SemaphoreType.DMA,
            pltpu.SemaphoreType.DMA,
            pltpu.SemaphoreType.DMA,
            pltpu.SemaphoreType.DMA,
            pltpu.SemaphoreType.DMA,
        ],
    )(outp, dest, pmax)


def _ffn_body(te_ref, ut_ref, x_ref, w1_ref, b1_ref, w2_ref, b2_ref, out_ref):
    i = pl.program_id(0)

    @pl.when(i < ut_ref[0])
    def _compute():
        x = x_ref[...]
        h = jnp.dot(x, w1_ref[0], preferred_element_type=jnp.float32)
        h = jnp.maximum(h + b1_ref[0], 0.0)
        o = jnp.dot(h, w2_ref[0], preferred_element_type=jnp.float32)
        out_ref[...] = o + b2_ref[0]


def _grouped_ffn(xs_padded, W1, b1, W2, b2, tile_expert, used_tiles):
    grid_spec = pltpu.PrefetchScalarGridSpec(
        num_scalar_prefetch=2,
        grid=(NT,),
        in_specs=[
            pl.BlockSpec((TM, D), lambda i, te, ut: (jnp.minimum(i, ut[0] - 1), 0)),
            pl.BlockSpec((1, D, F), lambda i, te, ut: (te[i], 0, 0)),
            pl.BlockSpec((1, 1, F), lambda i, te, ut: (te[i], 0, 0)),
            pl.BlockSpec((1, F, D), lambda i, te, ut: (te[i], 0, 0)),
            pl.BlockSpec((1, 1, D), lambda i, te, ut: (te[i], 0, 0)),
        ],
        out_specs=pl.BlockSpec(
            (TM, D), lambda i, te, ut: (jnp.minimum(i, ut[0] - 1), 0)),
    )
    return pl.pallas_call(
        _ffn_body,
        grid_spec=grid_spec,
        out_shape=jax.ShapeDtypeStruct((NPAD, D), jnp.float32),
    )(tile_expert, used_tiles, xs_padded, W1, b1.reshape(E, 1, F), W2,
      b2.reshape(E, 1, D))


def kernel(x, Ws, bs, W1, b1, W2, b2):
    bsz, seq, d_model = x.shape
    xf = x.reshape(-1, d_model)

    routes3, ranks3, pmax3, aux, starts2, te2, ut2 = _router(xf, Ws, bs)
    routes = routes3.reshape(N)
    ranks = ranks3.reshape(N)
    pmax = pmax3.reshape(N)
    starts = starts2.reshape(E)
    tile_expert = te2.reshape(NT)
    used_tiles = ut2.reshape(1)

    xs_padded, dest = _sc_scatter(xf, routes, ranks, starts)
    out_padded = _grouped_ffn(xs_padded, W1, b1, W2, b2, tile_expert, used_tiles)
    final = _sc_gather(out_padded, dest, pmax)

    return final.reshape(bsz, seq, d_model), aux[0, 0]


# speedup vs baseline: 1.4349x; 1.4349x over previous
import functools

import jax
import jax.numpy as jnp
from jax import lax
from jax.experimental import pallas as pl
from jax.experimental.pallas import tpu as pltpu
from jax.experimental.pallas import tpu_sc as plsc

N = 8192
D = 768
E = 64
F = 1024
RB = 512
NRB = N // RB
TM = 256
NPAD = 24576
NT = NPAD // TM

SC_CORES = 2
SC_SUBCORES = 16
NW = SC_CORES * SC_SUBCORES
TPW = N // NW
CB = 64
NCH = TPW // CB


def _router_body(x_ref, ws_ref, bs_ref, routes_ref, ranks_ref, pmax_ref,
                 aux_ref, starts_ref, te_ref, ut_ref,
                 tri_ref, counts_ref, colsum_ref):
    i = pl.program_id(0)

    @pl.when(i == 0)
    def _init():
        counts_ref[...] = jnp.zeros_like(counts_ref)
        colsum_ref[...] = jnp.zeros_like(colsum_ref)
        ri = lax.broadcasted_iota(jnp.int32, (RB, RB), 0)
        rj = lax.broadcasted_iota(jnp.int32, (RB, RB), 1)
        tri_ref[...] = (rj <= ri).astype(jnp.bfloat16)

    x = x_ref[...]
    logits = jnp.dot(x, ws_ref[...], preferred_element_type=jnp.float32)
    logits = logits + bs_ref[...]

    m = jnp.max(logits, axis=1, keepdims=True)
    p = jnp.exp(logits - m)
    s = jnp.sum(p, axis=1, keepdims=True)
    probs = p / s
    pmax = 1.0 / s[:, 0]

    iota_e = lax.broadcasted_iota(jnp.int32, (RB, E), 1)
    is_max = logits == m
    routes = jnp.min(jnp.where(is_max, iota_e, E), axis=1)

    onehot = (iota_e == routes[:, None]).astype(jnp.float32)

    csum = jnp.dot(tri_ref[...], onehot.astype(jnp.bfloat16),
                   preferred_element_type=jnp.float32)
    rank_in_blk = jnp.sum(csum * onehot, axis=1) - 1.0
    running = counts_ref[...]
    rank = rank_in_blk + jnp.sum(onehot * running, axis=1)

    cnt_blk = jnp.sum(onehot, axis=0, keepdims=True)
    counts_ref[...] = running + cnt_blk
    colsum_ref[...] = colsum_ref[...] + jnp.sum(probs, axis=0, keepdims=True)

    routes_ref[...] = routes.reshape(1, 1, RB)
    ranks_ref[...] = rank.astype(jnp.int32).reshape(1, 1, RB)
    pmax_ref[...] = pmax.reshape(1, 1, RB)

    @pl.when(i == NRB - 1)
    def _fin():
        counts = counts_ref[...]
        aux_ref[...] = ((E / N) * jnp.sum(counts * colsum_ref[...])
                        ).reshape(1, 1)
        padded = jnp.ceil(counts / TM) * TM
        ei = lax.broadcasted_iota(jnp.int32, (E, E), 0)
        ej = lax.broadcasted_iota(jnp.int32, (E, E), 1)
        triu = (ei <= ej).astype(jnp.bfloat16)
        ends = jnp.dot(padded.astype(jnp.bfloat16), triu,
                       preferred_element_type=jnp.float32)
        starts_ref[...] = (ends - padded).astype(jnp.int32)
        ends_last = jnp.max(ends, axis=1, keepdims=True)
        used = ends_last / TM
        ut_ref[...] = used.astype(jnp.int32)
        tgrid = lax.broadcasted_iota(jnp.int32, (NT, E), 0).astype(
            jnp.float32) * TM
        endsg = jnp.broadcast_to(ends, (NT, E))
        te_raw = jnp.sum((endsg <= tgrid).astype(jnp.float32), axis=1)
        te_last = jnp.minimum(
            jnp.sum((ends <= ends_last - TM).astype(jnp.float32)), E - 1.0)
        ti = lax.broadcasted_iota(jnp.int32, (NT,), 0).astype(jnp.float32)
        te = jnp.where(ti < used[0, 0], jnp.minimum(te_raw, E - 1.0), te_last)
        te_ref[...] = te.astype(jnp.int32).reshape(1, NT)


def _router(xf, Ws, bs):
    return pl.pallas_call(
        _router_body,
        grid=(NRB,),
        in_specs=[
            pl.BlockSpec((RB, D), lambda i: (i, 0)),
            pl.BlockSpec((D, E), lambda i: (0, 0)),
            pl.BlockSpec((1, E), lambda i: (0, 0)),
        ],
        out_specs=[
            pl.BlockSpec((1, 1, RB), lambda i: (i, 0, 0)),
            pl.BlockSpec((1, 1, RB), lambda i: (i, 0, 0)),
            pl.BlockSpec((1, 1, RB), lambda i: (i, 0, 0)),
            pl.BlockSpec((1, 1), lambda i: (0, 0)),
            pl.BlockSpec((1, E), lambda i: (0, 0)),
            pl.BlockSpec((1, NT), lambda i: (0, 0)),
            pl.BlockSpec((1, 1), lambda i: (0, 0)),
        ],
        out_shape=[
            jax.ShapeDtypeStruct((NRB, 1, RB), jnp.int32),
            jax.ShapeDtypeStruct((NRB, 1, RB), jnp.int32),
            jax.ShapeDtypeStruct((NRB, 1, RB), jnp.float32),
            jax.ShapeDtypeStruct((1, 1), jnp.float32),
            jax.ShapeDtypeStruct((1, E), jnp.int32),
            jax.ShapeDtypeStruct((1, NT), jnp.int32),
            jax.ShapeDtypeStruct((1, 1), jnp.int32),
        ],
        scratch_shapes=[
            pltpu.VMEM((RB, RB), jnp.bfloat16),
            pltpu.VMEM((1, E), jnp.float32),
            pltpu.VMEM((1, E), jnp.float32),
        ],
    )(xf, Ws, bs.reshape(1, E))


def _sc_mesh():
    return plsc.VectorSubcoreMesh(
        core_axis_name="c", subcore_axis_name="s",
        num_cores=SC_CORES, num_subcores=SC_SUBCORES)


_SC_PARAMS = pltpu.CompilerParams(needs_layout_passes=False)


def _sc_scatter_body(xf, routes, ranks, starts, xs, dest,
                     rows0, rows1, rr0, rr1, idx_v, starts_v,
                     sr0, sr1, sq0, sq1, sk0, sk1, sem):
    wid = lax.axis_index("s") * SC_CORES + lax.axis_index("c")
    wbase = wid * TPW
    rows = (rows0, rows1)
    rr = (rr0, rr1)
    srow = (sr0, sr1)
    srt = (sq0, sq1)
    srk = (sk0, sk1)
    pltpu.sync_copy(starts, starts_v)

    def start_in(c):
        b = c % 2
        base = wbase + c * CB
        return (
            pltpu.async_copy(xf.at[pl.ds(base, CB)], rows[b], srow[b]),
            pltpu.async_copy(routes.at[pl.ds(base, CB)], rr[b].at[0], srt[b]),
            pltpu.async_copy(ranks.at[pl.ds(base, CB)], rr[b].at[1], srk[b]),
        )

    h_in = start_in(0)
    for c in range(NCH):
        b = c % 2
        for h in h_in:
            h.wait()
        if c + 1 < NCH:
            h_in = start_in(c + 1)
        for k in range(CB // 16):
            r = rr[b][0, pl.ds(k * 16, 16)]
            s_r = plsc.load_gather(starts_v, [r])
            idx_v[c, pl.ds(k * 16, 16)] = s_r + rr[b][1, pl.ds(k * 16, 16)]
        pltpu.async_copy(rows[b], xs.at[idx_v.at[c]], sem).wait()
        pltpu.sync_copy(idx_v.at[c], dest.at[pl.ds(wbase + c * CB, CB)])


def _sc_scatter(xf, routes, ranks, starts):
    return pl.kernel(
        _sc_scatter_body,
        out_type=[
            jax.ShapeDtypeStruct((NPAD, D), jnp.float32),
            jax.ShapeDtypeStruct((N,), jnp.int32),
        ],
        mesh=_sc_mesh(),
        compiler_params=_SC_PARAMS,
        scratch_types=[
            pltpu.VMEM((CB, D), jnp.float32),
            pltpu.VMEM((CB, D), jnp.float32),
            pltpu.VMEM((2, CB), jnp.int32),
            pltpu.VMEM((2, CB), jnp.int32),
            pltpu.VMEM((NCH, CB), jnp.int32),
            pltpu.VMEM((E,), jnp.int32),
            pltpu.SemaphoreType.DMA,
            pltpu.SemaphoreType.DMA,
            pltpu.SemaphoreType.DMA,
            pltpu.SemaphoreType.DMA,
            pltpu.SemaphoreType.DMA,
            pltpu.SemaphoreType.DMA,
            pltpu.SemaphoreType.DMA,
        ],
    )(xf, routes, ranks, starts)


def _sc_gather_body(outp, dest, pmax, final,
                    rows_v, idx0, idx1, pm0, pm1,
                    si0, si1, sp0, sp1, sem):
    wid = lax.axis_index("s") * SC_CORES + lax.axis_index("c")
    wbase = wid * TPW
    idxb = (idx0, idx1)
    pmb = (pm0, pm1)
    sidx = (si0, si1)
    spm = (sp0, sp1)

    def start_idx(c):
        b = c % 2
        base = wbase + c * CB
        return (
            pltpu.async_copy(dest.at[pl.ds(base, CB)], idxb[b], sidx[b]),
            pltpu.async_copy(pmax.at[pl.ds(base, CB)], pmb[b], spm[b]),
        )

    h_in = start_idx(0)
    for c in range(NCH):
        b = c % 2
        for h in h_in:
            h.wait()
        if c + 1 < NCH:
            h_in = start_idx(c + 1)
        pltpu.async_copy(outp.at[idxb[b]], rows_v, sem).wait()

        def _scale(k, carry):
            pv = plsc.load_gather(pmb[b], [jnp.zeros((16,), jnp.int32) + k])
            for j in range(D // 16):
                rows_v[k, pl.ds(j * 16, 16)] = rows_v[k, pl.ds(j * 16, 16)] * pv
            return carry

        lax.fori_loop(0, CB, _scale, 0)
        pltpu.sync_copy(rows_v, final.at[pl.ds(wbase + c * CB, CB)])


def _sc_gather(outp, dest, pmax):
    return pl.kernel(
        _sc_gather_body,
        out_type=jax.ShapeDtypeStruct((N, D), jnp.float32),
        mesh=_sc_mesh(),
        compiler_params=_SC_PARAMS,
        scratch_types=[
            pltpu.VMEM((CB, D), jnp.float32),
            pltpu.VMEM((CB,), jnp.int32),
            pltpu.VMEM((CB,), jnp.int32),
            pltpu.VMEM((CB,), jnp.float32),
            pltpu.VMEM((CB,), jnp.float32),
            pltpu.SemaphoreType.DMA,
            pltpu.SemaphoreType.DMA,
            pltpu.SemaphoreType.DMA,
            pltpu.SemaphoreType.DMA,
            pltpu.SemaphoreType.DMA,
        ],
    )(outp, dest, pmax)


def _ffn_body(te_ref, ut_ref, x_ref, w1_ref, b1_ref, w2_ref, b2_ref, out_ref):
    i = pl.program_id(0)

    @pl.when(i < ut_ref[0])
    def _compute():
        x = x_ref[...]
        h = jnp.dot(x, w1_ref[0], preferred_element_type=jnp.float32)
        h = jnp.maximum(h + b1_ref[0], 0.0)
        o = jnp.dot(h, w2_ref[0], preferred_element_type=jnp.float32)
        out_ref[...] = o + b2_ref[0]


def _grouped_ffn(xs_padded, W1, b1, W2, b2, tile_expert, used_tiles):
    grid_spec = pltpu.PrefetchScalarGridSpec(
        num_scalar_prefetch=2,
        grid=(NT,),
        in_specs=[
            pl.BlockSpec((TM, D), lambda i, te, ut: (jnp.minimum(i, ut[0] - 1), 0)),
            pl.BlockSpec((1, D, F), lambda i, te, ut: (te[i], 0, 0)),
            pl.BlockSpec((1, 1, F), lambda i, te, ut: (te[i], 0, 0)),
            pl.BlockSpec((1, F, D), lambda i, te, ut: (te[i], 0, 0)),
            pl.BlockSpec((1, 1, D), lambda i, te, ut: (te[i], 0, 0)),
        ],
        out_specs=pl.BlockSpec(
            (TM, D), lambda i, te, ut: (jnp.minimum(i, ut[0] - 1), 0)),
    )
    return pl.pallas_call(
        _ffn_body,
        grid_spec=grid_spec,
        out_shape=jax.ShapeDtypeStruct((NPAD, D), jnp.float32),
    )(tile_expert, used_tiles, xs_padded, W1, b1.reshape(E, 1, F), W2,
      b2.reshape(E, 1, D))


def kernel(x, Ws, bs, W1, b1, W2, b2):
    bsz, seq, d_model = x.shape
    xf = x.reshape(-1, d_model)

    routes3, ranks3, pmax3, aux, starts2, te2, ut2 = _router(xf, Ws, bs)
    routes = routes3.reshape(N)
    ranks = ranks3.reshape(N)
    pmax = pmax3.reshape(N)
    starts = starts2.reshape(E)
    tile_expert = te2.reshape(NT)
    used_tiles = ut2.reshape(1)

    xs_padded, dest = _sc_scatter(xf, routes, ranks, starts)
    out_padded = _grouped_ffn(xs_padded, W1, b1, W2, b2, tile_expert, used_tiles)
    final = _sc_gather(out_padded, dest, pmax)

    return final.reshape(bsz, seq, d_model), aux[0, 0]
